# Initial kernel scaffold; baseline (speedup 1.0000x reference)
import jax
import jax.numpy as jnp
from jax.experimental import pallas as pl


def kernel(input, target):
    raise NotImplementedError("write your pallas kernel here")


# speedup vs baseline: 35.6332x; 35.6332x over previous
import jax
import jax.numpy as jnp
from jax.experimental import pallas as pl
from jax.experimental.pallas import tpu as pltpu

_N = 524288
_ROWS = 512
_COLS = 1024


def _body(x_ref, t_ref, out_ref):
    x = x_ref[...]
    t = t_ref[...]
    bits = jax.lax.bitcast_convert_type(x, jnp.uint32)
    u = jnp.where(bits >= jnp.uint32(0x80000000), ~bits, bits | jnp.uint32(0x80000000))
    is_pos = t > 0.0
    u = jnp.where(is_pos, jnp.uint32(0), u)
    pos_num_f = jnp.sum(jnp.where(is_pos, 1.0, 0.0))
    k = (pos_num_f * 3.0).astype(jnp.int32)

    def step(_, carry):
        lo, hi = carry
        d = hi - lo
        mid = lo + (d >> jnp.uint32(1)) + (d & jnp.uint32(1))
        cnt = jnp.sum(jnp.where(u >= mid, 1, 0))
        ok = cnt >= k
        return (jnp.where(ok, mid, lo), jnp.where(ok, hi, mid - jnp.uint32(1)))

    lo, _ = jax.lax.fori_loop(
        0, 32, step, (jnp.uint32(0), jnp.uint32(0xFFFFFFFF))
    )
    v = lo
    vb = jnp.where(v >= jnp.uint32(0x80000000), v ^ jnp.uint32(0x80000000), ~v)
    x_v = jax.lax.bitcast_convert_type(vb, jnp.float32)

    log1p_exp = jnp.log1p(jnp.exp(-jnp.abs(x)))
    softplus = jnp.maximum(x, 0.0) + log1p_exp
    gt = u > v
    count_gt = jnp.sum(jnp.where(gt, 1, 0))
    sum_gt = jnp.sum(jnp.where(gt, softplus, 0.0))
    pos_sum = jnp.sum(jnp.where(is_pos, softplus - x * t, 0.0))
    sp_v = jnp.maximum(x_v, 0.0) + jnp.log1p(jnp.exp(-jnp.abs(x_v)))
    tie_sum = (k - count_gt).astype(jnp.float32) * sp_v
    total = pos_num_f + k.astype(jnp.float32)
    out_ref[0, 0] = (pos_sum + sum_gt + tie_sum) / total


def kernel(input, target):
    x2 = input.reshape(_ROWS, _COLS)
    t2 = target.reshape(_ROWS, _COLS)
    out = pl.pallas_call(
        _body,
        out_shape=jax.ShapeDtypeStruct((1, 1), jnp.float32),
        out_specs=pl.BlockSpec(memory_space=pltpu.SMEM),
    )(x2, t2)
    return out[0, 0]
